# baseline (speedup 1.0000x reference)
import jax
import jax.numpy as jnp
from jax.experimental import pallas as pl
from jax.experimental.pallas import tpu as pltpu

_G = 8
_NBUF = 8


def _body(rows_ref, he_ref, hn_ref, gst_ref, gn_ref, x_ref,
          dw_hbm, db_ref, uw_hbm, o_ref, dwb_ref, uwb_ref, sem):
    m = pl.program_id(0)
    j = pl.program_id(1)
    NJ = pl.num_programs(1)
    S = x_ref.shape[1]
    g = m * NJ + j
    total = pl.num_programs(0) * NJ

    def copies(gg, slot):
        mm = gg // NJ
        jj = gg % NJ
        e = he_ref[mm, jj]
        return (pltpu.make_async_copy(dw_hbm.at[mm, e], dwb_ref.at[slot],
                                      sem.at[slot, 0]),
                pltpu.make_async_copy(uw_hbm.at[mm, e], uwb_ref.at[slot],
                                      sem.at[slot, 1]))

    def issue(gg):
        mm = gg // NJ
        jj = gg % NJ

        @pl.when(jj < hn_ref[mm, 0])
        def _():
            for c in copies(gg, gg % _NBUF):
                c.start()

    @pl.when(g == 0)
    def _():
        for l in range(_NBUF - 1):
            issue(l)

    @pl.when(g + _NBUF - 1 < total)
    def _():
        issue(g + _NBUF - 1)

    @pl.when(j < hn_ref[m, 0])
    def _():
        slot = g % _NBUF
        for c in copies(g, slot):
            c.wait()

        t0 = gst_ref[m, j]
        t1 = t0 + gn_ref[m, j]

        def down(t):
            rows = tuple(rows_ref[m, t * _G + i] for i in range(_G))
            xt = jnp.concatenate([x_ref[r] for r in rows], axis=0)
            z = jnp.dot(xt, dwb_ref[slot],
                        preferred_element_type=jnp.float32)
            z = z + db_ref[0, 0, 0][None, :]
            return z * jax.nn.sigmoid(z), rows

        def up_store(z, rows):
            u = jnp.dot(z, uwb_ref[slot],
                        preferred_element_type=jnp.float32)
            for i in range(_G):
                o_ref[0, rows[i]] = u[i * S:(i + 1) * S]

        def step(t, carry):
            nxt = down(t)
            up_store(*carry)
            return nxt

        last = jax.lax.fori_loop(t0 + 1, t1, step, down(t0))
        up_store(*last)


def _routing(expert_index, N, NG):
    M, B = expert_index.shape
    iN = jnp.arange(N, dtype=jnp.int32)
    oh = jax.nn.one_hot(expert_index, N, dtype=jnp.int32)
    counts = jnp.sum(oh, axis=1)
    gsz = (counts + _G - 1) // _G
    estart = jnp.cumsum(counts, axis=1) - counts
    gbefore = jnp.cumsum(gsz, axis=1) - gsz

    ohcum = jnp.cumsum(oh, axis=1)
    within = jnp.einsum('mbn,mbn->mb', ohcum, oh) - 1
    pos = jnp.einsum('mbn,mn->mb', oh, estart) + within
    pos_oh = jax.nn.one_hot(pos, B, dtype=jnp.int32)
    order = jnp.einsum('mbp,b->mp', pos_oh, jnp.arange(B, dtype=jnp.int32))

    g = jnp.arange(NG)[None, None, :]
    in_e = ((g >= gbefore[:, :, None])
            & (g < (gbefore + gsz)[:, :, None])).astype(jnp.int32)
    gidx = jnp.arange(NG, dtype=jnp.int32)[None, :]
    cnt_g = jnp.einsum('mng,mn->mg', in_e, counts)
    gb_g = jnp.einsum('mng,mn->mg', in_e, gbefore)
    es_g = jnp.einsum('mng,mn->mg', in_e, estart)
    qc = jnp.clip(cnt_g - (gidx - gb_g) * _G, 0, _G)

    sI = jnp.arange(NG * _G, dtype=jnp.int32)[None, :] % _G
    qc_r = jnp.repeat(qc, _G, axis=1)
    posg = (jnp.repeat(es_g, _G, axis=1)
            + (jnp.arange(NG * _G)[None, :] // _G - jnp.repeat(gb_g, _G, axis=1)) * _G
            + jnp.where(qc_r > 0, sI % jnp.maximum(qc_r, 1), 0))
    posg = jnp.clip(posg, 0, B - 1)
    rows = jnp.einsum('mib,mb->mi', jax.nn.one_hot(posg, B, dtype=jnp.int32),
                      order).astype(jnp.int32)

    hit = (counts > 0).astype(jnp.int32)
    hn = jnp.sum(hit, axis=1, keepdims=True).astype(jnp.int32)
    hrank = jnp.cumsum(hit, axis=1) - hit
    jj = jnp.arange(N, dtype=jnp.int32)
    ind = (hit[:, :, None] * (hrank[:, :, None] == jj[None, None, :])
           ).astype(jnp.int32)
    he = jnp.einsum('mnj,n->mj', ind, iN).astype(jnp.int32)
    gst = jnp.einsum('mnj,mn->mj', ind, gbefore).astype(jnp.int32)
    gn = jnp.einsum('mnj,mn->mj', ind, gsz).astype(jnp.int32)
    return rows, he, hn, gst, gn


def kernel(x, expert_index, down_w, down_b, up_w):
    B, S, C = x.shape
    M, N, _, D = down_w.shape
    NG = (B + (_G - 1) * N) // _G

    rows, he, hn, gst, gn = _routing(expert_index, N, NG)
    db4 = down_b.reshape(M, N, 1, D)

    grid_spec = pltpu.PrefetchScalarGridSpec(
        num_scalar_prefetch=5,
        grid=(M, N),
        in_specs=[
            pl.BlockSpec((B, S, C), lambda m, j, rw, he, hn, gs, gn: (0, 0, 0)),
            pl.BlockSpec(memory_space=pl.ANY),
            pl.BlockSpec((1, 1, 1, D),
                         lambda m, j, rw, he, hn, gs, gn: (m, he[m, j], 0, 0)),
            pl.BlockSpec(memory_space=pl.ANY),
        ],
        out_specs=pl.BlockSpec((1, B, S, C),
                               lambda m, j, rw, he, hn, gs, gn: (m, 0, 0, 0)),
        scratch_shapes=[
            pltpu.VMEM((_NBUF, C, D), jnp.float32),
            pltpu.VMEM((_NBUF, D, C), jnp.float32),
            pltpu.SemaphoreType.DMA((_NBUF, 2)),
        ],
    )

    out = pl.pallas_call(
        _body,
        grid_spec=grid_spec,
        out_shape=jax.ShapeDtypeStruct((M, B, S, C), jnp.float32),
        compiler_params=pltpu.CompilerParams(
            dimension_semantics=("arbitrary", "arbitrary"),
        ),
    )(rows, he, hn, gst, gn, x, down_w, db4, up_w)
    return out


# speedup vs baseline: 1.3979x; 1.3979x over previous
import jax
import jax.numpy as jnp
from jax.experimental import pallas as pl
from jax.experimental.pallas import tpu as pltpu

_G = 8
_K = 8
_NBUF = 4


def _body(tp_ref, ge_ref, rows_ref, x_ref, dw_hbm, db_ref, uw_hbm, o_ref,
          dwb_ref, uwb_ref, sem):
    m = pl.program_id(0)
    ck = pl.program_id(1)
    NC = pl.num_programs(1)
    S = x_ref.shape[1]
    g = m * NC + ck
    total = pl.num_programs(0) * NC

    def copies(gg, slot):
        mm = gg // NC
        cc = gg % NC
        return (pltpu.make_async_copy(dw_hbm.at[mm, pl.ds(cc * _K, _K)],
                                      dwb_ref.at[slot], sem.at[slot, 0]),
                pltpu.make_async_copy(uw_hbm.at[mm, pl.ds(cc * _K, _K)],
                                      uwb_ref.at[slot], sem.at[slot, 1]))

    @pl.when(g == 0)
    def _():
        for l in range(_NBUF - 1):
            for c in copies(l, l):
                c.start()

    @pl.when(g + _NBUF - 1 < total)
    def _():
        gg = g + _NBUF - 1
        for c in copies(gg, gg % _NBUF):
            c.start()

    slot = g % _NBUF
    for c in copies(g, slot):
        c.wait()

    t0 = tp_ref[m, ck]
    t1 = tp_ref[m, ck + 1]

    def down(t):
        e_local = ge_ref[m, t] - ck * _K
        rows = tuple(rows_ref[m, t * _G + i] for i in range(_G))
        xt = jnp.concatenate([x_ref[r] for r in rows], axis=0)
        z = jnp.dot(xt, dwb_ref[slot, e_local],
                    preferred_element_type=jnp.float32)
        z = z + db_ref[0, e_local, 0][None, :]
        return z * jax.nn.sigmoid(z), e_local, rows

    def up_store(z, e_local, rows):
        u = jnp.dot(z, uwb_ref[slot, e_local],
                    preferred_element_type=jnp.float32)
        for i in range(_G):
            o_ref[0, rows[i]] = u[i * S:(i + 1) * S]

    @pl.when(t1 > t0)
    def _():
        def step(t, carry):
            nxt = down(t)
            up_store(*carry)
            return nxt

        last = jax.lax.fori_loop(t0 + 1, t1, step, down(t0))
        up_store(*last)


def _routing(expert_index, N, NG):
    M, B = expert_index.shape
    iN = jnp.arange(N, dtype=jnp.int32)
    oh = jax.nn.one_hot(expert_index, N, dtype=jnp.int32)
    counts = jnp.sum(oh, axis=1)
    gsz = (counts + _G - 1) // _G
    estart = jnp.cumsum(counts, axis=1) - counts
    gbefore = jnp.cumsum(gsz, axis=1) - gsz
    ngroups = jnp.sum(gsz, axis=1)

    ohcum = jnp.cumsum(oh, axis=1)
    within = jnp.einsum('mbn,mbn->mb', ohcum, oh) - 1
    pos = jnp.einsum('mbn,mn->mb', oh, estart) + within
    pos_oh = jax.nn.one_hot(pos, B, dtype=jnp.int32)
    order = jnp.einsum('mbp,b->mp', pos_oh, jnp.arange(B, dtype=jnp.int32))

    g = jnp.arange(NG)[None, None, :]
    in_e = ((g >= gbefore[:, :, None])
            & (g < (gbefore + gsz)[:, :, None])).astype(jnp.int32)
    ge = jnp.einsum('mng,n->mg', in_e, iN).astype(jnp.int32)
    gidx = jnp.arange(NG, dtype=jnp.int32)[None, :]

    cnt_g = jnp.einsum('mng,mn->mg', in_e, counts)
    gb_g = jnp.einsum('mng,mn->mg', in_e, gbefore)
    es_g = jnp.einsum('mng,mn->mg', in_e, estart)
    qc = jnp.clip(cnt_g - (gidx - gb_g) * _G, 0, _G)

    sI = jnp.arange(NG * _G, dtype=jnp.int32)[None, :] % _G
    qc_r = jnp.repeat(qc, _G, axis=1)
    posg = (jnp.repeat(es_g, _G, axis=1)
            + (jnp.arange(NG * _G)[None, :] // _G - jnp.repeat(gb_g, _G, axis=1)) * _G
            + jnp.where(qc_r > 0, sI % jnp.maximum(qc_r, 1), 0))
    posg = jnp.clip(posg, 0, B - 1)
    rows = jnp.einsum('mib,mb->mi', jax.nn.one_hot(posg, B, dtype=jnp.int32),
                      order).astype(jnp.int32)

    gb_ext = jnp.concatenate([gbefore, ngroups[:, None]], axis=1)
    tp = gb_ext[:, ::_K].astype(jnp.int32)
    return tp, ge, rows


def kernel(x, expert_index, down_w, down_b, up_w):
    B, S, C = x.shape
    M, N, _, D = down_w.shape
    NG = (B + (_G - 1) * N) // _G
    NC = N // _K

    tp, ge, rows = _routing(expert_index, N, NG)
    db4 = down_b.reshape(M, N, 1, D)

    grid_spec = pltpu.PrefetchScalarGridSpec(
        num_scalar_prefetch=3,
        grid=(M, NC),
        in_specs=[
            pl.BlockSpec((B, S, C), lambda m, ck, tp, ge, rw: (0, 0, 0)),
            pl.BlockSpec(memory_space=pl.ANY),
            pl.BlockSpec((1, _K, 1, D), lambda m, ck, tp, ge, rw: (m, ck, 0, 0)),
            pl.BlockSpec(memory_space=pl.ANY),
        ],
        out_specs=pl.BlockSpec((1, B, S, C), lambda m, ck, tp, ge, rw: (m, 0, 0, 0)),
        scratch_shapes=[
            pltpu.VMEM((_NBUF, _K, C, D), jnp.float32),
            pltpu.VMEM((_NBUF, _K, D, C), jnp.float32),
            pltpu.SemaphoreType.DMA((_NBUF, 2)),
        ],
    )

    out = pl.pallas_call(
        _body,
        grid_spec=grid_spec,
        out_shape=jax.ShapeDtypeStruct((M, B, S, C), jnp.float32),
        compiler_params=pltpu.CompilerParams(
            dimension_semantics=("arbitrary", "arbitrary"),
        ),
    )(tp, ge, rows, x, down_w, db4, up_w)
    return out
